# baseline (speedup 1.0000x reference)
import functools

import jax
import jax.numpy as jnp
from jax import lax
from jax.experimental import pallas as pl
from jax.experimental.pallas import tpu as pltpu
from jax.experimental.pallas import tpu_sc as plsc

L = 16
NS = 16
NC = 2
BH = 128
NB = BH // L
K = 32


def _rup(a, b):
    return -(-a // b) * b


GC = 8


@functools.partial(jax.jit, static_argnums=(4, 5))
def _sc_segment(xT2, src_g, dst_g, w_g, n_out_pad, n_groups):
    mesh = plsc.VectorSubcoreMesh(core_axis_name="c", subcore_axis_name="s")
    nz = n_out_pad // NS

    def body(xT2_hbm, src_hbm, dst_hbm, w_hbm, out_hbm,
             src_vm, dst_vm, w_vm, rows_vm, zb_vm, acc_sh, sem):
        c = lax.axis_index("c")
        s = lax.axis_index("s")
        row0 = s * nz
        zv = jnp.zeros((L,), jnp.float32)
        for r in range(16):
            for j in range(NB):
                zb_vm[r, pl.ds(j * L, L)] = zv

        def zloop(g, carry):
            pltpu.sync_copy(zb_vm, acc_sh.at[pl.ds(row0 + g * 16, 16)])
            return carry
        lax.fori_loop(0, nz // 16, zloop, 0)
        plsc.subcore_barrier()

        def gloop(og, carry):
            pltpu.sync_copy(src_hbm.at[c, s, og], src_vm)
            pltpu.sync_copy(dst_hbm.at[s, og], dst_vm)
            pltpu.sync_copy(w_hbm.at[s, og], w_vm)
            for g in range(GC):
                pltpu.async_copy(xT2_hbm.at[src_vm.at[g]], rows_vm, sem).wait()
                for h in range(K // L):
                    wv16 = w_vm[g, pl.ds(h * L, L)]
                    for e16 in range(L):
                        e = h * L + e16
                        wv = jnp.full((L,), wv16[e16])
                        for j in range(NB):
                            rows_vm[e, pl.ds(j * L, L)] = rows_vm[e, pl.ds(j * L, L)] * wv
                pltpu.sync_copy(rows_vm, acc_sh.at[dst_vm.at[g]], add=True)
            return carry
        lax.fori_loop(0, n_groups, gloop, 0)
        plsc.subcore_barrier()

        def dloop(g, carry):
            pltpu.sync_copy(acc_sh.at[pl.ds(row0 + g * 16, 16)],
                            out_hbm.at[c, pl.ds(row0 + g * 16, 16)])
            return carry
        lax.fori_loop(0, nz // 16, dloop, 0)

    return pl.kernel(
        body,
        out_type=jax.ShapeDtypeStruct((NC, n_out_pad, BH), jnp.float32),
        mesh=mesh,
        scratch_types=[
            pltpu.VMEM((GC, K), jnp.int32),
            pltpu.VMEM((GC, K), jnp.int32),
            pltpu.VMEM((GC, K), jnp.float32),
            pltpu.VMEM((K, BH), jnp.float32),
            pltpu.VMEM((16, BH), jnp.float32),
            pltpu.VMEM_SHARED((n_out_pad, BH), jnp.float32),
            pltpu.SemaphoreType.DMA,
        ],
    )(xT2, src_g, dst_g, w_g)


def _softplus(r):
    return jnp.log1p(jnp.exp(r))


def _kl(mu, sigma):
    return jnp.sum(-jnp.log(sigma) + 0.5 * (sigma ** 2 + mu ** 2) - 0.5)


def _pack(hT):
    n = hT.shape[0]
    hp = jnp.pad(hT, ((0, 0), (0, 2 * BH - hT.shape[1])))
    return hp.reshape(n, NC, BH).swapaxes(0, 1).reshape(NC * n, BH)


def _unpack(out2, n_out):
    return jnp.concatenate([out2[0, :n_out], out2[1, :n_out]], axis=1)[:, :200]


def _sparse_layer(hT, src, dst, wmu, wrho, bmu, brho, ew, eb, n_in, n_out):
    sw = _softplus(wrho)
    sb = _softplus(brho)
    w = wmu + sw * ew
    b = bmu + sb * eb
    kl = _kl(wmu, sw) + _kl(bmu, sb)

    E = src.shape[0]
    Ep = _rup(E, NS * GC * K)
    pad = Ep - E
    srcp = jnp.pad(src, (0, pad))
    dstp = jnp.pad(dst, (0, pad))
    wp = jnp.pad(w, (0, pad))
    n_groups = Ep // (NS * GC * K)
    src_g = jnp.stack([srcp, srcp + n_in]).reshape(NC, NS, n_groups, GC, K)
    dst_g = dstp.reshape(NS, n_groups, GC, K)
    w_g = wp.reshape(NS, n_groups, GC, K)

    n_out_pad = _rup(n_out, 256)
    out2 = _sc_segment(_pack(hT), src_g, dst_g, w_g, n_out_pad, n_groups)
    hT_out = _unpack(out2, n_out) + b[:, None]
    return hT_out, kl


def _bn_t(hT, g, b):
    m = jnp.mean(hT, axis=1, keepdims=True)
    v = jnp.var(hT, axis=1, keepdims=True)
    xn = (hT - m) / jnp.sqrt(v + 1e-5)
    if g is None:
        return xn
    return xn * g[:, None] + b[:, None]


_SIZES = [10000, 2500, 625, 156, 39, 10, 3]


def kernel(x, sl0_src, sl0_dst, sl0_wmu, sl0_wrho, sl0_bmu, sl0_brho, sl0_ew, sl0_eb, sl1_src, sl1_dst, sl1_wmu, sl1_wrho, sl1_bmu, sl1_brho, sl1_ew, sl1_eb, sl2_src, sl2_dst, sl2_wmu, sl2_wrho, sl2_bmu, sl2_brho, sl2_ew, sl2_eb, sl3_src, sl3_dst, sl3_wmu, sl3_wrho, sl3_bmu, sl3_brho, sl3_ew, sl3_eb, sl4_src, sl4_dst, sl4_wmu, sl4_wrho, sl4_bmu, sl4_brho, sl4_ew, sl4_eb, sl5_src, sl5_dst, sl5_wmu, sl5_wrho, sl5_bmu, sl5_brho, sl5_ew, sl5_eb, sl6_src, sl6_dst, sl6_wmu, sl6_wrho, sl6_bmu, sl6_brho, sl6_ew, sl6_eb, sp1_src, sp1_dst, sp1_wmu, sp1_wrho, sp1_bmu, sp1_brho, sp1_ew, sp1_eb, sp2_src, sp2_dst, sp2_wmu, sp2_wrho, sp2_bmu, sp2_brho, sp2_ew, sp2_eb, sp3_src, sp3_dst, sp3_wmu, sp3_wrho, sp3_bmu, sp3_brho, sp3_ew, sp3_eb, sp4_src, sp4_dst, sp4_wmu, sp4_wrho, sp4_bmu, sp4_brho, sp4_ew, sp4_eb, sp5_src, sp5_dst, sp5_wmu, sp5_wrho, sp5_bmu, sp5_brho, sp5_ew, sp5_eb, sp6_src, sp6_dst, sp6_wmu, sp6_wrho, sp6_bmu, sp6_brho, sp6_ew, sp6_eb, bn0_g, bn0_b, bn1_g, bn1_b, bn2_g, bn2_b, bn3_g, bn3_b, bn4_g, bn4_b, bn5_g, bn5_b):
    kw = dict(locals())
    hT = x.reshape(x.shape[0], -1).T

    hT, kl_tot = _sparse_layer(hT, sl0_src, sl0_dst, sl0_wmu, sl0_wrho,
                               sl0_bmu, sl0_brho, sl0_ew, sl0_eb,
                               _SIZES[0], _SIZES[0])
    hT = jax.nn.relu(_bn_t(hT, bn0_g, bn0_b))
    for i in range(1, 7):
        hT, kl = _sparse_layer(hT, kw[f"sp{i}_src"], kw[f"sp{i}_dst"],
                               kw[f"sp{i}_wmu"], kw[f"sp{i}_wrho"],
                               kw[f"sp{i}_bmu"], kw[f"sp{i}_brho"],
                               kw[f"sp{i}_ew"], kw[f"sp{i}_eb"],
                               _SIZES[i - 1], _SIZES[i])
        kl_tot = kl_tot + kl
        hT, kl = _sparse_layer(hT, kw[f"sl{i}_src"], kw[f"sl{i}_dst"],
                               kw[f"sl{i}_wmu"], kw[f"sl{i}_wrho"],
                               kw[f"sl{i}_bmu"], kw[f"sl{i}_brho"],
                               kw[f"sl{i}_ew"], kw[f"sl{i}_eb"],
                               _SIZES[i], _SIZES[i])
        kl_tot = kl_tot + kl
        if i < 6:
            hT = jax.nn.relu(_bn_t(hT, kw[f"bn{i}_g"], kw[f"bn{i}_b"]))
        else:
            hT = _bn_t(hT, None, None)
    return hT.T, kl_tot


# speedup vs baseline: 1.7605x; 1.7605x over previous
import functools

import jax
import jax.numpy as jnp
from jax import lax
from jax.experimental import pallas as pl
from jax.experimental.pallas import tpu as pltpu
from jax.experimental.pallas import tpu_sc as plsc

L = 16
NS = 16
NC = 2
BH = 128
NB = BH // L
K = 32


def _rup(a, b):
    return -(-a // b) * b


GC = 4


@functools.partial(jax.jit, static_argnums=(4, 5))
def _sc_segment(xT2, src_g, dst_g, w_g, n_out_pad, n_groups):
    mesh = plsc.VectorSubcoreMesh(core_axis_name="c", subcore_axis_name="s")
    nz = n_out_pad // NS

    def body(xT2_hbm, src_hbm, dst_hbm, w_hbm, out_hbm,
             src_vm, dst_vm, w_vm, rows_vm, zb_vm, acc_sh, sem, ssem):
        c = lax.axis_index("c")
        s = lax.axis_index("s")
        row0 = s * nz
        zv = jnp.zeros((L,), jnp.float32)
        for r in range(16):
            for j in range(NB):
                zb_vm[r, pl.ds(j * L, L)] = zv

        def zloop(g, carry):
            pltpu.sync_copy(zb_vm, acc_sh.at[pl.ds(row0 + g * 16, 16)])
            return carry
        lax.fori_loop(0, nz // 16, zloop, 0)
        plsc.subcore_barrier()

        def gloop(og, carry):
            pltpu.sync_copy(src_hbm.at[c, s, og], src_vm)
            pltpu.sync_copy(dst_hbm.at[s, og], dst_vm)
            pltpu.sync_copy(w_hbm.at[s, og], w_vm)
            gds = [pltpu.async_copy(xT2_hbm.at[src_vm.at[g]], rows_vm.at[g], sem)
                   for g in range(GC)]
            sds = []
            for g in range(GC):
                gds[g].wait()
                for h in range(K // L):
                    wv16 = w_vm[g, pl.ds(h * L, L)]
                    for e16 in range(L):
                        e = h * L + e16
                        wv = jnp.full((L,), wv16[e16])
                        for j in range(NB):
                            rows_vm[g, e, pl.ds(j * L, L)] = rows_vm[g, e, pl.ds(j * L, L)] * wv
                sds.append(pltpu.async_copy(rows_vm.at[g], acc_sh.at[dst_vm.at[g]],
                                            ssem, add=True))
            for d in sds:
                d.wait()
            return carry
        lax.fori_loop(0, n_groups, gloop, 0)
        plsc.subcore_barrier()

        def dloop(g, carry):
            pltpu.sync_copy(acc_sh.at[pl.ds(row0 + g * 16, 16)],
                            out_hbm.at[c, pl.ds(row0 + g * 16, 16)])
            return carry
        lax.fori_loop(0, nz // 16, dloop, 0)

    return pl.kernel(
        body,
        out_type=jax.ShapeDtypeStruct((NC, n_out_pad, BH), jnp.float32),
        mesh=mesh,
        scratch_types=[
            pltpu.VMEM((GC, K), jnp.int32),
            pltpu.VMEM((GC, K), jnp.int32),
            pltpu.VMEM((GC, K), jnp.float32),
            pltpu.VMEM((GC, K, BH), jnp.float32),
            pltpu.VMEM((16, BH), jnp.float32),
            pltpu.VMEM_SHARED((n_out_pad, BH), jnp.float32),
            pltpu.SemaphoreType.DMA,
            pltpu.SemaphoreType.DMA,
        ],
    )(xT2, src_g, dst_g, w_g)


def _softplus(r):
    return jnp.log1p(jnp.exp(r))


def _kl(mu, sigma):
    return jnp.sum(-jnp.log(sigma) + 0.5 * (sigma ** 2 + mu ** 2) - 0.5)


def _pack(hT):
    n = hT.shape[0]
    hp = jnp.pad(hT, ((0, 0), (0, 2 * BH - hT.shape[1])))
    return hp.reshape(n, NC, BH).swapaxes(0, 1).reshape(NC * n, BH)


def _unpack(out2, n_out):
    return jnp.concatenate([out2[0, :n_out], out2[1, :n_out]], axis=1)[:, :200]


def _sparse_layer(hT, src, dst, wmu, wrho, bmu, brho, ew, eb, n_in, n_out):
    sw = _softplus(wrho)
    sb = _softplus(brho)
    w = wmu + sw * ew
    b = bmu + sb * eb
    kl = _kl(wmu, sw) + _kl(bmu, sb)

    E = src.shape[0]
    Ep = _rup(E, NS * GC * K)
    pad = Ep - E
    srcp = jnp.pad(src, (0, pad))
    dstp = jnp.pad(dst, (0, pad))
    wp = jnp.pad(w, (0, pad))
    n_groups = Ep // (NS * GC * K)
    src_g = jnp.stack([srcp, srcp + n_in]).reshape(NC, NS, n_groups, GC, K)
    dst_g = dstp.reshape(NS, n_groups, GC, K)
    w_g = wp.reshape(NS, n_groups, GC, K)

    n_out_pad = _rup(n_out, 256)
    out2 = _sc_segment(_pack(hT), src_g, dst_g, w_g, n_out_pad, n_groups)
    hT_out = _unpack(out2, n_out) + b[:, None]
    return hT_out, kl


def _bn_t(hT, g, b):
    m = jnp.mean(hT, axis=1, keepdims=True)
    v = jnp.var(hT, axis=1, keepdims=True)
    xn = (hT - m) / jnp.sqrt(v + 1e-5)
    if g is None:
        return xn
    return xn * g[:, None] + b[:, None]


_SIZES = [10000, 2500, 625, 156, 39, 10, 3]


def kernel(x, sl0_src, sl0_dst, sl0_wmu, sl0_wrho, sl0_bmu, sl0_brho, sl0_ew, sl0_eb, sl1_src, sl1_dst, sl1_wmu, sl1_wrho, sl1_bmu, sl1_brho, sl1_ew, sl1_eb, sl2_src, sl2_dst, sl2_wmu, sl2_wrho, sl2_bmu, sl2_brho, sl2_ew, sl2_eb, sl3_src, sl3_dst, sl3_wmu, sl3_wrho, sl3_bmu, sl3_brho, sl3_ew, sl3_eb, sl4_src, sl4_dst, sl4_wmu, sl4_wrho, sl4_bmu, sl4_brho, sl4_ew, sl4_eb, sl5_src, sl5_dst, sl5_wmu, sl5_wrho, sl5_bmu, sl5_brho, sl5_ew, sl5_eb, sl6_src, sl6_dst, sl6_wmu, sl6_wrho, sl6_bmu, sl6_brho, sl6_ew, sl6_eb, sp1_src, sp1_dst, sp1_wmu, sp1_wrho, sp1_bmu, sp1_brho, sp1_ew, sp1_eb, sp2_src, sp2_dst, sp2_wmu, sp2_wrho, sp2_bmu, sp2_brho, sp2_ew, sp2_eb, sp3_src, sp3_dst, sp3_wmu, sp3_wrho, sp3_bmu, sp3_brho, sp3_ew, sp3_eb, sp4_src, sp4_dst, sp4_wmu, sp4_wrho, sp4_bmu, sp4_brho, sp4_ew, sp4_eb, sp5_src, sp5_dst, sp5_wmu, sp5_wrho, sp5_bmu, sp5_brho, sp5_ew, sp5_eb, sp6_src, sp6_dst, sp6_wmu, sp6_wrho, sp6_bmu, sp6_brho, sp6_ew, sp6_eb, bn0_g, bn0_b, bn1_g, bn1_b, bn2_g, bn2_b, bn3_g, bn3_b, bn4_g, bn4_b, bn5_g, bn5_b):
    kw = dict(locals())
    hT = x.reshape(x.shape[0], -1).T

    hT, kl_tot = _sparse_layer(hT, sl0_src, sl0_dst, sl0_wmu, sl0_wrho,
                               sl0_bmu, sl0_brho, sl0_ew, sl0_eb,
                               _SIZES[0], _SIZES[0])
    hT = jax.nn.relu(_bn_t(hT, bn0_g, bn0_b))
    for i in range(1, 7):
        hT, kl = _sparse_layer(hT, kw[f"sp{i}_src"], kw[f"sp{i}_dst"],
                               kw[f"sp{i}_wmu"], kw[f"sp{i}_wrho"],
                               kw[f"sp{i}_bmu"], kw[f"sp{i}_brho"],
                               kw[f"sp{i}_ew"], kw[f"sp{i}_eb"],
                               _SIZES[i - 1], _SIZES[i])
        kl_tot = kl_tot + kl
        hT, kl = _sparse_layer(hT, kw[f"sl{i}_src"], kw[f"sl{i}_dst"],
                               kw[f"sl{i}_wmu"], kw[f"sl{i}_wrho"],
                               kw[f"sl{i}_bmu"], kw[f"sl{i}_brho"],
                               kw[f"sl{i}_ew"], kw[f"sl{i}_eb"],
                               _SIZES[i], _SIZES[i])
        kl_tot = kl_tot + kl
        if i < 6:
            hT = jax.nn.relu(_bn_t(hT, kw[f"bn{i}_g"], kw[f"bn{i}_b"]))
        else:
            hT = _bn_t(hT, None, None)
    return hT.T, kl_tot
